# baseline (speedup 1.0000x reference)
import functools

import jax
import jax.numpy as jnp
from jax import lax
from jax.experimental import pallas as pl
from jax.experimental.pallas import tpu as pltpu
from jax.experimental.pallas import tpu_sc as plsc

_NC = 2
_NS = 16
_NW = _NC * _NS
_L = 16

_N = 10000
_D = 128
_H = 128
_E = 320000
_P = 160000

_NPAD = 10240
_EPAD = 327680
_PP = 163840
_LPAD = 2 * _PP
_CH = 128
_EPW = _EPAD // _NW
_ECH = _EPW // _CH
_TCH = _EPAD // _CH
_N0 = 40
_N1 = 2 * _ECH - _N0
_CHL = 64
_LPW = _LPAD // _NW
_LCH = _LPW // _CHL
_TLCH = _LPAD // _CHL
_L0 = 120
_L1 = 2 * _LCH - _L0
_RPT = _NPAD // _NS

_R = 512
_NB = _NPAD // _R

_mesh = plsc.VectorSubcoreMesh(
    core_axis_name="c", subcore_axis_name="s", num_cores=_NC, num_subcores=_NS)


def _wid():
    return lax.axis_index("s") * _NC + lax.axis_index("c")


def _deg_body(dst_hbm, out_hbm, idx_v, ones_v, zch_v, deg_sh):
    c = lax.axis_index("c")
    s = lax.axis_index("s")
    w = _wid()

    def _fill_z(i, carry):
        zch_v[pl.ds(i * _L, _L)] = jnp.zeros((_L,), jnp.float32)
        return carry

    lax.fori_loop(0, 2048 // _L, _fill_z, 0)

    def _fill_one(i, carry):
        ones_v[pl.ds(i * _L, _L)] = jnp.ones((_L,), jnp.float32)
        return carry

    lax.fori_loop(0, _CH // _L, _fill_one, 0)
    pltpu.sync_copy(dst_hbm.at[pl.ds(w * _ECH, _ECH)], idx_v)

    @pl.when(s == 0)
    def _():
        def _z(i, carry):
            pltpu.sync_copy(zch_v, deg_sh.at[pl.ds(i * 2048, 2048)])
            return carry
        lax.fori_loop(0, _NPAD // 2048, _z, 0)

    plsc.subcore_barrier()

    def _chunk(i, carry):
        pltpu.sync_copy(ones_v, deg_sh.at[idx_v.at[i]], add=True)
        return carry

    lax.fori_loop(0, _ECH, _chunk, 0)
    plsc.subcore_barrier()

    @pl.when(s == 0)
    def _():
        pltpu.sync_copy(deg_sh, out_hbm.at[pl.ds(c * _NPAD, _NPAD)])


_deg_call = functools.partial(
    pl.kernel,
    out_type=jax.ShapeDtypeStruct((_NC * _NPAD,), jnp.float32),
    mesh=_mesh,
    scratch_types=[
        pltpu.VMEM((_ECH, _CH), jnp.int32),
        pltpu.VMEM((_CH,), jnp.float32),
        pltpu.VMEM((2048,), jnp.float32),
        pltpu.VMEM_SHARED((_NPAD,), jnp.float32),
    ],
)(_deg_body)


def _agg_body(hs_hbm, src_hbm, dst_hbm, out_hbm, isrc_v, id0, id1, rows0,
              rows1, acc_sh, gs0, gs1):
    c = lax.axis_index("c")
    s = lax.axis_index("s")
    w = _wid()

    def _zero_row(r, carry):
        for j in range(_D // _L):
            rows0[r, pl.ds(j * _L, _L)] = jnp.zeros((_L,), jnp.float32)
        return carry

    lax.fori_loop(0, _CH, _zero_row, 0)
    for k in range(_RPT // _CH):
        pltpu.sync_copy(rows0, acc_sh.at[pl.ds(s * _RPT + k * _CH, _CH)])
    base = jnp.where(c == 0, s * _N0, _N0 * _NS + s * _N1)
    nch = jnp.where(c == 0, _N0, _N1)
    pltpu.sync_copy(src_hbm.at[pl.ds(base, _N1)], isrc_v)
    plsc.subcore_barrier()

    pltpu.async_copy(hs_hbm.at[isrc_v.at[0]], rows0, gs0)
    pltpu.async_copy(dst_hbm.at[base], id0, gs0)

    def _drain(rows, idx, sem):
        pltpu.make_async_copy(hs_hbm.at[isrc_v.at[0]], rows, sem).wait()
        pltpu.make_async_copy(dst_hbm.at[0], idx, sem).wait()

    def _body(ii, carry):
        i0 = ii * 2
        i1 = i0 + 1
        _drain(rows0, id0, gs0)
        pltpu.async_copy(hs_hbm.at[isrc_v.at[i1]], rows1, gs1)
        pltpu.async_copy(dst_hbm.at[base + i1], id1, gs1)
        pltpu.sync_copy(rows0, acc_sh.at[id0], add=True)
        _drain(rows1, id1, gs1)

        @pl.when(i1 + 1 < nch)
        def _():
            pltpu.async_copy(hs_hbm.at[isrc_v.at[i1 + 1]], rows0, gs0)
            pltpu.async_copy(dst_hbm.at[base + i1 + 1], id0, gs0)

        pltpu.sync_copy(rows1, acc_sh.at[id1], add=True)
        return carry

    lax.fori_loop(0, nch // 2, _body, 0)
    plsc.subcore_barrier()

    for k in range(_RPT // _CH):
        r0 = s * _RPT + k * _CH
        pltpu.sync_copy(acc_sh.at[pl.ds(r0, _CH)],
                        out_hbm.at[pl.ds(c * _NPAD + r0, _CH)])


_agg_call = functools.partial(
    pl.kernel,
    out_type=jax.ShapeDtypeStruct((_NC * _NPAD, _D), jnp.float32),
    mesh=_mesh,
    scratch_types=[
        pltpu.VMEM((_N1, _CH), jnp.int32),
        pltpu.VMEM((_CH,), jnp.int32),
        pltpu.VMEM((_CH,), jnp.int32),
        pltpu.VMEM((_CH, _D), jnp.float32),
        pltpu.VMEM((_CH, _D), jnp.float32),
        pltpu.VMEM_SHARED((_NPAD, _D), jnp.float32),
        pltpu.SemaphoreType.DMA,
        pltpu.SemaphoreType.DMA,
    ],
)(_agg_body)


def _link_body(zw_hbm, z_hbm, ai_hbm, bi_hbm, out_hbm, ia_v, ib_v, za0, zb0,
               za1, zb1, dots_v, gs0, gs1):
    c = lax.axis_index("c")
    s = lax.axis_index("s")
    base = jnp.where(c == 0, s * _L0, _L0 * _NS + s * _L1)
    nl = jnp.where(c == 0, _L0, _L1)
    pltpu.sync_copy(ai_hbm.at[pl.ds(base, _L1)], ia_v)
    pltpu.sync_copy(bi_hbm.at[pl.ds(base, _L1)], ib_v)
    pltpu.async_copy(zw_hbm.at[ia_v.at[0]], za0, gs0)
    pltpu.async_copy(z_hbm.at[ib_v.at[0]], zb0, gs0)

    def _compute(za_v, zb_v, i):
        def _pair(p, carry2):
            p0 = p * 2
            p1 = p0 + 1
            acc0 = za_v[p0, pl.ds(0, _L)] * zb_v[p0, pl.ds(0, _L)]
            acc1 = za_v[p1, pl.ds(0, _L)] * zb_v[p1, pl.ds(0, _L)]
            for j in range(1, 2 * _H // _L):
                sl = pl.ds(j * _L, _L)
                acc0 = acc0 + za_v[p0, sl] * zb_v[p0, sl]
                acc1 = acc1 + za_v[p1, sl] * zb_v[p1, sl]
            dots_v[p0, :] = acc0
            dots_v[p1, :] = acc1
            return carry2

        lax.fori_loop(0, _CHL // 2, _pair, 0)
        pltpu.sync_copy(dots_v, out_hbm.at[pl.ds((base + i) * _CHL, _CHL)])

    def _body(ii, carry):
        i0 = ii * 2
        i1 = i0 + 1
        pltpu.make_async_copy(zw_hbm.at[ia_v.at[0]], za0, gs0).wait()
        pltpu.make_async_copy(zw_hbm.at[ia_v.at[0]], zb0, gs0).wait()
        pltpu.async_copy(zw_hbm.at[ia_v.at[i1]], za1, gs1)
        pltpu.async_copy(z_hbm.at[ib_v.at[i1]], zb1, gs1)
        _compute(za0, zb0, i0)
        pltpu.make_async_copy(zw_hbm.at[ia_v.at[0]], za1, gs1).wait()
        pltpu.make_async_copy(zw_hbm.at[ia_v.at[0]], zb1, gs1).wait()

        @pl.when(i1 + 1 < nl)
        def _():
            pltpu.async_copy(zw_hbm.at[ia_v.at[i1 + 1]], za0, gs0)
            pltpu.async_copy(z_hbm.at[ib_v.at[i1 + 1]], zb0, gs0)

        _compute(za1, zb1, i1)
        return carry

    lax.fori_loop(0, nl // 2, _body, 0)


_link_call = functools.partial(
    pl.kernel,
    out_type=jax.ShapeDtypeStruct((_LPAD, _L), jnp.float32),
    mesh=_mesh,
    scratch_types=[
        pltpu.VMEM((_L1, _CHL), jnp.int32),
        pltpu.VMEM((_L1, _CHL), jnp.int32),
        pltpu.VMEM((_CHL, 2 * _H), jnp.float32),
        pltpu.VMEM((_CHL, 2 * _H), jnp.float32),
        pltpu.VMEM((_CHL, 2 * _H), jnp.float32),
        pltpu.VMEM((_CHL, 2 * _H), jnp.float32),
        pltpu.VMEM((_CHL, _L), jnp.float32),
        pltpu.SemaphoreType.DMA,
        pltpu.SemaphoreType.DMA,
    ],
)(_link_body)


def _dinv_of(d0_ref, d1_ref):
    d = d0_ref[0] + d1_ref[0] + 1.0
    return lax.rsqrt(d)


def _s1_body(x_ref, d0_ref, d1_ref, w1_ref, hs_ref):
    dinv = _dinv_of(d0_ref, d1_ref)
    h = jnp.dot(x_ref[...], w1_ref[...], preferred_element_type=jnp.float32)
    hs_ref[...] = h * dinv


def _s2_body(a0_ref, a1_ref, hs1_ref, d0_ref, d1_ref, b1_ref, w2_ref,
             h1_ref, hs2_ref):
    dinv = _dinv_of(d0_ref, d1_ref)
    t = (a0_ref[...] + a1_ref[...] + hs1_ref[...]) * dinv + b1_ref[...]
    h1 = jnp.maximum(t, 0.0)
    h1_ref[...] = h1
    hs2_ref[...] = jnp.dot(h1, w2_ref[...],
                           preferred_element_type=jnp.float32) * dinv


def _s3_body(a0_ref, a1_ref, hs2_ref, d0_ref, d1_ref, b2_ref, h1_ref, wp_ref,
             z_ref, zw_ref):
    dinv = _dinv_of(d0_ref, d1_ref)
    t = (a0_ref[...] + a1_ref[...] + hs2_ref[...]) * dinv + b2_ref[...]
    h2 = jnp.maximum(t, 0.0)
    zb = jnp.concatenate([h1_ref[...], h2], axis=1)
    z_ref[...] = zb
    zw_ref[...] = zb * wp_ref[...]


def _loss_body(dots_ref, bp_ref, out_ref):
    b = pl.program_id(0)
    blk = dots_ref[...]
    lane = lax.broadcasted_iota(jnp.int32, (128, 8), 0)
    grp = lax.broadcasted_iota(jnp.int32, (128, 8), 1)
    sel = jnp.where(lane // _L == grp, 1.0, 0.0)
    logit = jnp.dot(blk, sel, preferred_element_type=jnp.float32) + bp_ref[0, 0]
    row = lax.broadcasted_iota(jnp.int32, (_R, 8), 0)
    col = lax.broadcasted_iota(jnp.int32, (_R, 8), 1)
    pi = (b * _R + row) * 8 + col
    is_pos = pi < _P
    valid = is_pos | ((pi >= _PP) & (pi < _PP + _P))
    tgt = jnp.where(is_pos, 1.0, 0.0)
    term = (jnp.maximum(logit, 0.0) - logit * tgt
            + jnp.log1p(jnp.exp(-jnp.abs(logit))))
    term = jnp.where(valid, term, 0.0)

    @pl.when(b == 0)
    def _():
        out_ref[...] = jnp.zeros((1, 1), jnp.float32)

    out_ref[...] += jnp.sum(term).reshape(1, 1) / (2.0 * _P)


def _row_spec():
    return pl.BlockSpec((_R, _D), lambda i: (i, 0))


def _deg_spec():
    return pl.BlockSpec((1, _R, 1), lambda i: (i, 0, 0))


def _full_spec(shape):
    return pl.BlockSpec(shape, lambda i: tuple(0 for _ in shape))


def kernel(x, ei, pos, neg, W1, b1, W2, b2, Wp, bp):
    f32 = jnp.float32
    i32 = jnp.int32

    pad_idx = _NPAD - 1
    xp = jnp.concatenate([x, jnp.zeros((_NPAD - _N, _D), f32)], axis=0)
    srcp = jnp.concatenate([ei[0], jnp.full((_EPAD - _E,), pad_idx, i32)])
    dstp = jnp.concatenate([ei[1], jnp.full((_EPAD - _E,), pad_idx, i32)])
    srcp = srcp.reshape(_TCH, _CH)
    dstp = dstp.reshape(_TCH, _CH)
    fillp = jnp.full((_PP - _P,), pad_idx, i32)
    ai = jnp.concatenate([pos[0], fillp, neg[0], fillp]).reshape(_TLCH, _CHL)
    bi = jnp.concatenate([pos[1], fillp, neg[1], fillp]).reshape(_TLCH, _CHL)
    b1r = b1.reshape(1, _H)
    b2r = b2.reshape(1, _H)
    wpr = Wp.reshape(1, 2 * _H)
    bpr = bp.reshape(1, 1)

    degp = _deg_call(dstp).reshape(_NC, _NB, _R, 1)
    d0, d1 = degp[0], degp[1]

    hs1 = pl.pallas_call(
        _s1_body,
        grid=(_NB,),
        in_specs=[_row_spec(), _deg_spec(), _deg_spec(), _full_spec((_D, _H))],
        out_specs=_row_spec(),
        out_shape=jax.ShapeDtypeStruct((_NPAD, _H), f32),
    )(xp, d0, d1, W1)
    acc1 = _agg_call(hs1, srcp, dstp).reshape(_NC, _NPAD, _H)

    h1, hs2 = pl.pallas_call(
        _s2_body,
        grid=(_NB,),
        in_specs=[_row_spec(), _row_spec(), _row_spec(), _deg_spec(),
                  _deg_spec(), _full_spec((1, _H)), _full_spec((_H, _H))],
        out_specs=[_row_spec(), _row_spec()],
        out_shape=[jax.ShapeDtypeStruct((_NPAD, _H), f32),
                   jax.ShapeDtypeStruct((_NPAD, _H), f32)],
    )(acc1[0], acc1[1], hs1, d0, d1, b1r, W2)
    acc2 = _agg_call(hs2, srcp, dstp).reshape(_NC, _NPAD, _H)

    z, zw = pl.pallas_call(
        _s3_body,
        grid=(_NB,),
        in_specs=[_row_spec(), _row_spec(), _row_spec(), _deg_spec(),
                  _deg_spec(), _full_spec((1, _H)), _row_spec(),
                  _full_spec((1, 2 * _H))],
        out_specs=[pl.BlockSpec((_R, 2 * _H), lambda i: (i, 0)),
                   pl.BlockSpec((_R, 2 * _H), lambda i: (i, 0))],
        out_shape=[jax.ShapeDtypeStruct((_NPAD, 2 * _H), f32),
                   jax.ShapeDtypeStruct((_NPAD, 2 * _H), f32)],
    )(acc2[0], acc2[1], hs2, d0, d1, b2r, h1, wpr)

    dots = _link_call(zw, z, ai, bi)
    dots2d = dots.reshape(_LPAD * _L // 128, 128)

    res = pl.pallas_call(
        _loss_body,
        grid=(dots2d.shape[0] // _R,),
        in_specs=[pl.BlockSpec((_R, 128), lambda i: (i, 0)),
                  _full_spec((1, 1))],
        out_specs=pl.BlockSpec((1, 1), lambda i: (0, 0)),
        out_shape=jax.ShapeDtypeStruct((1, 1), f32),
    )(dots2d, bpr)
    return res[0, 0]


# speedup vs baseline: 1.0681x; 1.0681x over previous
import functools

import jax
import jax.numpy as jnp
from jax import lax
from jax.experimental import pallas as pl
from jax.experimental.pallas import tpu as pltpu
from jax.experimental.pallas import tpu_sc as plsc

_NC = 2
_NS = 16
_NW = _NC * _NS
_L = 16

_N = 10000
_D = 128
_H = 128
_E = 320000
_P = 160000

_NPAD = 10240
_EPAD = 327680
_PP = 163840
_LPAD = 2 * _PP
_CH = 128
_EPW = _EPAD // _NW
_ECH = _EPW // _CH
_TCH = _EPAD // _CH
_N0 = 120
_N1 = 2 * _ECH - _N0
_NMX = max(_N0, _N1)
_CHL = 64
_LPW = _LPAD // _NW
_LCH = _LPW // _CHL
_TLCH = _LPAD // _CHL
_L0 = 200
_L1 = 2 * _LCH - _L0
_LMX = max(_L0, _L1)
_RPT = _NPAD // _NS

_R = 512
_NB = _NPAD // _R

_mesh = plsc.VectorSubcoreMesh(
    core_axis_name="c", subcore_axis_name="s", num_cores=_NC, num_subcores=_NS)


def _wid():
    return lax.axis_index("s") * _NC + lax.axis_index("c")


def _deg_body(dst_hbm, out_hbm, idx_v, ones_v, zch_v, deg_sh):
    c = lax.axis_index("c")
    s = lax.axis_index("s")
    w = _wid()

    def _fill_z(i, carry):
        zch_v[pl.ds(i * _L, _L)] = jnp.zeros((_L,), jnp.float32)
        return carry

    lax.fori_loop(0, 2048 // _L, _fill_z, 0)

    def _fill_one(i, carry):
        ones_v[pl.ds(i * _L, _L)] = jnp.ones((_L,), jnp.float32)
        return carry

    lax.fori_loop(0, _CH // _L, _fill_one, 0)
    pltpu.sync_copy(dst_hbm.at[pl.ds(w * _ECH, _ECH)], idx_v)

    @pl.when(s == 0)
    def _():
        def _z(i, carry):
            pltpu.sync_copy(zch_v, deg_sh.at[pl.ds(i * 2048, 2048)])
            return carry
        lax.fori_loop(0, _NPAD // 2048, _z, 0)

    plsc.subcore_barrier()

    def _chunk(i, carry):
        pltpu.sync_copy(ones_v, deg_sh.at[idx_v.at[i]], add=True)
        return carry

    lax.fori_loop(0, _ECH, _chunk, 0)
    plsc.subcore_barrier()

    @pl.when(s == 0)
    def _():
        pltpu.sync_copy(deg_sh, out_hbm.at[pl.ds(c * _NPAD, _NPAD)])


_deg_call = functools.partial(
    pl.kernel,
    out_type=jax.ShapeDtypeStruct((_NC * _NPAD,), jnp.float32),
    mesh=_mesh,
    scratch_types=[
        pltpu.VMEM((_ECH, _CH), jnp.int32),
        pltpu.VMEM((_CH,), jnp.float32),
        pltpu.VMEM((2048,), jnp.float32),
        pltpu.VMEM_SHARED((_NPAD,), jnp.float32),
    ],
)(_deg_body)


def _agg_body(hs_hbm, src_hbm, dst_hbm, out_hbm, isrc_v, id0, id1, rows0,
              rows1, acc_sh, gs0, gs1):
    c = lax.axis_index("c")
    s = lax.axis_index("s")
    w = _wid()

    def _zero_row(r, carry):
        for j in range(_D // _L):
            rows0[r, pl.ds(j * _L, _L)] = jnp.zeros((_L,), jnp.float32)
        return carry

    lax.fori_loop(0, _CH, _zero_row, 0)
    for k in range(_RPT // _CH):
        pltpu.sync_copy(rows0, acc_sh.at[pl.ds(s * _RPT + k * _CH, _CH)])
    base = jnp.where(c == 0, s * _N0, _N0 * _NS + s * _N1)
    nch = jnp.where(c == 0, _N0, _N1)
    pltpu.sync_copy(src_hbm.at[pl.ds(base, _NMX)], isrc_v)
    plsc.subcore_barrier()

    pltpu.async_copy(hs_hbm.at[isrc_v.at[0]], rows0, gs0)
    pltpu.async_copy(dst_hbm.at[base], id0, gs0)

    def _drain(rows, idx, sem):
        pltpu.make_async_copy(hs_hbm.at[isrc_v.at[0]], rows, sem).wait()
        pltpu.make_async_copy(dst_hbm.at[0], idx, sem).wait()

    def _body(ii, carry):
        i0 = ii * 2
        i1 = i0 + 1
        _drain(rows0, id0, gs0)
        pltpu.async_copy(hs_hbm.at[isrc_v.at[i1]], rows1, gs1)
        pltpu.async_copy(dst_hbm.at[base + i1], id1, gs1)
        pltpu.sync_copy(rows0, acc_sh.at[id0], add=True)
        _drain(rows1, id1, gs1)

        @pl.when(i1 + 1 < nch)
        def _():
            pltpu.async_copy(hs_hbm.at[isrc_v.at[i1 + 1]], rows0, gs0)
            pltpu.async_copy(dst_hbm.at[base + i1 + 1], id0, gs0)

        pltpu.sync_copy(rows1, acc_sh.at[id1], add=True)
        return carry

    lax.fori_loop(0, nch // 2, _body, 0)
    plsc.subcore_barrier()

    for k in range(_RPT // _CH):
        r0 = s * _RPT + k * _CH
        pltpu.sync_copy(acc_sh.at[pl.ds(r0, _CH)],
                        out_hbm.at[pl.ds(c * _NPAD + r0, _CH)])


_agg_call = functools.partial(
    pl.kernel,
    out_type=jax.ShapeDtypeStruct((_NC * _NPAD, _D), jnp.float32),
    mesh=_mesh,
    scratch_types=[
        pltpu.VMEM((_NMX, _CH), jnp.int32),
        pltpu.VMEM((_CH,), jnp.int32),
        pltpu.VMEM((_CH,), jnp.int32),
        pltpu.VMEM((_CH, _D), jnp.float32),
        pltpu.VMEM((_CH, _D), jnp.float32),
        pltpu.VMEM_SHARED((_NPAD, _D), jnp.float32),
        pltpu.SemaphoreType.DMA,
        pltpu.SemaphoreType.DMA,
    ],
)(_agg_body)


def _link_body(zw_hbm, z_hbm, ai_hbm, bi_hbm, out_hbm, ia_v, ib_v, za0, zb0,
               za1, zb1, dots_v, gs0, gs1):
    c = lax.axis_index("c")
    s = lax.axis_index("s")
    base = jnp.where(c == 0, s * _L0, _L0 * _NS + s * _L1)
    nl = jnp.where(c == 0, _L0, _L1)
    pltpu.sync_copy(ai_hbm.at[pl.ds(base, _LMX)], ia_v)
    pltpu.sync_copy(bi_hbm.at[pl.ds(base, _LMX)], ib_v)
    pltpu.async_copy(zw_hbm.at[ia_v.at[0]], za0, gs0)
    pltpu.async_copy(z_hbm.at[ib_v.at[0]], zb0, gs0)

    def _compute(za_v, zb_v, i):
        def _pair(p, carry2):
            p0 = p * 2
            p1 = p0 + 1
            acc0 = za_v[p0, pl.ds(0, _L)] * zb_v[p0, pl.ds(0, _L)]
            acc1 = za_v[p1, pl.ds(0, _L)] * zb_v[p1, pl.ds(0, _L)]
            for j in range(1, 2 * _H // _L):
                sl = pl.ds(j * _L, _L)
                acc0 = acc0 + za_v[p0, sl] * zb_v[p0, sl]
                acc1 = acc1 + za_v[p1, sl] * zb_v[p1, sl]
            dots_v[p0, :] = acc0
            dots_v[p1, :] = acc1
            return carry2

        lax.fori_loop(0, _CHL // 2, _pair, 0)
        pltpu.sync_copy(dots_v, out_hbm.at[pl.ds((base + i) * _CHL, _CHL)])

    def _body(ii, carry):
        i0 = ii * 2
        i1 = i0 + 1
        pltpu.make_async_copy(zw_hbm.at[ia_v.at[0]], za0, gs0).wait()
        pltpu.make_async_copy(zw_hbm.at[ia_v.at[0]], zb0, gs0).wait()
        pltpu.async_copy(zw_hbm.at[ia_v.at[i1]], za1, gs1)
        pltpu.async_copy(z_hbm.at[ib_v.at[i1]], zb1, gs1)
        _compute(za0, zb0, i0)
        pltpu.make_async_copy(zw_hbm.at[ia_v.at[0]], za1, gs1).wait()
        pltpu.make_async_copy(zw_hbm.at[ia_v.at[0]], zb1, gs1).wait()

        @pl.when(i1 + 1 < nl)
        def _():
            pltpu.async_copy(zw_hbm.at[ia_v.at[i1 + 1]], za0, gs0)
            pltpu.async_copy(z_hbm.at[ib_v.at[i1 + 1]], zb0, gs0)

        _compute(za1, zb1, i1)
        return carry

    lax.fori_loop(0, nl // 2, _body, 0)


_link_call = functools.partial(
    pl.kernel,
    out_type=jax.ShapeDtypeStruct((_LPAD, _L), jnp.float32),
    mesh=_mesh,
    scratch_types=[
        pltpu.VMEM((_LMX, _CHL), jnp.int32),
        pltpu.VMEM((_LMX, _CHL), jnp.int32),
        pltpu.VMEM((_CHL, 2 * _H), jnp.float32),
        pltpu.VMEM((_CHL, 2 * _H), jnp.float32),
        pltpu.VMEM((_CHL, 2 * _H), jnp.float32),
        pltpu.VMEM((_CHL, 2 * _H), jnp.float32),
        pltpu.VMEM((_CHL, _L), jnp.float32),
        pltpu.SemaphoreType.DMA,
        pltpu.SemaphoreType.DMA,
    ],
)(_link_body)


def _dinv_of(d0_ref, d1_ref):
    d = d0_ref[0] + d1_ref[0] + 1.0
    return lax.rsqrt(d)


def _s1_body(x_ref, d0_ref, d1_ref, w1_ref, hs_ref):
    dinv = _dinv_of(d0_ref, d1_ref)
    h = jnp.dot(x_ref[...], w1_ref[...], preferred_element_type=jnp.float32)
    hs_ref[...] = h * dinv


def _s2_body(a0_ref, a1_ref, hs1_ref, d0_ref, d1_ref, b1_ref, w2_ref,
             h1_ref, hs2_ref):
    dinv = _dinv_of(d0_ref, d1_ref)
    t = (a0_ref[...] + a1_ref[...] + hs1_ref[...]) * dinv + b1_ref[...]
    h1 = jnp.maximum(t, 0.0)
    h1_ref[...] = h1
    hs2_ref[...] = jnp.dot(h1, w2_ref[...],
                           preferred_element_type=jnp.float32) * dinv


def _s3_body(a0_ref, a1_ref, hs2_ref, d0_ref, d1_ref, b2_ref, h1_ref, wp_ref,
             z_ref, zw_ref):
    dinv = _dinv_of(d0_ref, d1_ref)
    t = (a0_ref[...] + a1_ref[...] + hs2_ref[...]) * dinv + b2_ref[...]
    h2 = jnp.maximum(t, 0.0)
    zb = jnp.concatenate([h1_ref[...], h2], axis=1)
    z_ref[...] = zb
    zw_ref[...] = zb * wp_ref[...]


def _loss_body(dots_ref, bp_ref, out_ref):
    b = pl.program_id(0)
    blk = dots_ref[...]
    lane = lax.broadcasted_iota(jnp.int32, (128, 8), 0)
    grp = lax.broadcasted_iota(jnp.int32, (128, 8), 1)
    sel = jnp.where(lane // _L == grp, 1.0, 0.0)
    logit = jnp.dot(blk, sel, preferred_element_type=jnp.float32) + bp_ref[0, 0]
    row = lax.broadcasted_iota(jnp.int32, (_R, 8), 0)
    col = lax.broadcasted_iota(jnp.int32, (_R, 8), 1)
    pi = (b * _R + row) * 8 + col
    is_pos = pi < _P
    valid = is_pos | ((pi >= _PP) & (pi < _PP + _P))
    tgt = jnp.where(is_pos, 1.0, 0.0)
    term = (jnp.maximum(logit, 0.0) - logit * tgt
            + jnp.log1p(jnp.exp(-jnp.abs(logit))))
    term = jnp.where(valid, term, 0.0)

    @pl.when(b == 0)
    def _():
        out_ref[...] = jnp.zeros((1, 1), jnp.float32)

    out_ref[...] += jnp.sum(term).reshape(1, 1) / (2.0 * _P)


def _row_spec():
    return pl.BlockSpec((_R, _D), lambda i: (i, 0))


def _deg_spec():
    return pl.BlockSpec((1, _R, 1), lambda i: (i, 0, 0))


def _full_spec(shape):
    return pl.BlockSpec(shape, lambda i: tuple(0 for _ in shape))


def kernel(x, ei, pos, neg, W1, b1, W2, b2, Wp, bp):
    f32 = jnp.float32
    i32 = jnp.int32

    pad_idx = _NPAD - 1
    xp = jnp.concatenate([x, jnp.zeros((_NPAD - _N, _D), f32)], axis=0)
    epad2 = (_TCH + abs(_N0 - _N1)) * _CH
    srcp = jnp.concatenate([ei[0], jnp.full((epad2 - _E,), pad_idx, i32)])
    dstp = jnp.concatenate([ei[1], jnp.full((epad2 - _E,), pad_idx, i32)])
    srcp = srcp.reshape(-1, _CH)
    dstp = dstp.reshape(-1, _CH)
    fillp = jnp.full((_PP - _P,), pad_idx, i32)
    lslack = jnp.full((abs(_L0 - _L1) * _CHL,), pad_idx, i32)
    ai = jnp.concatenate([pos[0], fillp, neg[0], fillp, lslack]).reshape(-1, _CHL)
    bi = jnp.concatenate([pos[1], fillp, neg[1], fillp, lslack]).reshape(-1, _CHL)
    b1r = b1.reshape(1, _H)
    b2r = b2.reshape(1, _H)
    wpr = Wp.reshape(1, 2 * _H)
    bpr = bp.reshape(1, 1)

    degp = _deg_call(dstp).reshape(_NC, _NB, _R, 1)
    d0, d1 = degp[0], degp[1]

    hs1 = pl.pallas_call(
        _s1_body,
        grid=(_NB,),
        in_specs=[_row_spec(), _deg_spec(), _deg_spec(), _full_spec((_D, _H))],
        out_specs=_row_spec(),
        out_shape=jax.ShapeDtypeStruct((_NPAD, _H), f32),
    )(xp, d0, d1, W1)
    acc1 = _agg_call(hs1, srcp, dstp).reshape(_NC, _NPAD, _H)

    h1, hs2 = pl.pallas_call(
        _s2_body,
        grid=(_NB,),
        in_specs=[_row_spec(), _row_spec(), _row_spec(), _deg_spec(),
                  _deg_spec(), _full_spec((1, _H)), _full_spec((_H, _H))],
        out_specs=[_row_spec(), _row_spec()],
        out_shape=[jax.ShapeDtypeStruct((_NPAD, _H), f32),
                   jax.ShapeDtypeStruct((_NPAD, _H), f32)],
    )(acc1[0], acc1[1], hs1, d0, d1, b1r, W2)
    acc2 = _agg_call(hs2, srcp, dstp).reshape(_NC, _NPAD, _H)

    z, zw = pl.pallas_call(
        _s3_body,
        grid=(_NB,),
        in_specs=[_row_spec(), _row_spec(), _row_spec(), _deg_spec(),
                  _deg_spec(), _full_spec((1, _H)), _row_spec(),
                  _full_spec((1, 2 * _H))],
        out_specs=[pl.BlockSpec((_R, 2 * _H), lambda i: (i, 0)),
                   pl.BlockSpec((_R, 2 * _H), lambda i: (i, 0))],
        out_shape=[jax.ShapeDtypeStruct((_NPAD, 2 * _H), f32),
                   jax.ShapeDtypeStruct((_NPAD, 2 * _H), f32)],
    )(acc2[0], acc2[1], hs2, d0, d1, b2r, h1, wpr)

    dots = _link_call(zw, z, ai, bi)
    dots2d = dots.reshape(_LPAD * _L // 128, 128)

    res = pl.pallas_call(
        _loss_body,
        grid=(dots2d.shape[0] // _R,),
        in_specs=[pl.BlockSpec((_R, 128), lambda i: (i, 0)),
                  _full_spec((1, 1))],
        out_specs=pl.BlockSpec((1, 1), lambda i: (0, 0)),
        out_shape=jax.ShapeDtypeStruct((1, 1), f32),
    )(dots2d, bpr)
    return res[0, 0]


# speedup vs baseline: 1.1101x; 1.0393x over previous
import functools

import jax
import jax.numpy as jnp
from jax import lax
from jax.experimental import pallas as pl
from jax.experimental.pallas import tpu as pltpu
from jax.experimental.pallas import tpu_sc as plsc

_NC = 2
_NS = 16
_NW = _NC * _NS
_L = 16

_N = 10000
_D = 128
_H = 128
_E = 320000
_P = 160000

_NPAD = 10240
_EPAD = 327680
_PP = 163840
_LPAD = 2 * _PP
_CH = 128
_EPW = _EPAD // _NW
_ECH = _EPW // _CH
_TCH = _EPAD // _CH
_N0 = 80
_N1 = 2 * _ECH - _N0
_NMX = max(_N0, _N1)
_CHL = 64
_LPW = _LPAD // _NW
_LCH = _LPW // _CHL
_TLCH = _LPAD // _CHL
_L0 = 160
_L1 = 2 * _LCH - _L0
_LMX = max(_L0, _L1)
_RPT = _NPAD // _NS

_R = 512
_NB = _NPAD // _R

_mesh = plsc.VectorSubcoreMesh(
    core_axis_name="c", subcore_axis_name="s", num_cores=_NC, num_subcores=_NS)


def _wid():
    return lax.axis_index("s") * _NC + lax.axis_index("c")


def _deg_body(dst_hbm, out_hbm, idx_v, ones_v, zch_v, deg_sh):
    c = lax.axis_index("c")
    s = lax.axis_index("s")
    w = _wid()

    def _fill_z(i, carry):
        zch_v[pl.ds(i * _L, _L)] = jnp.zeros((_L,), jnp.float32)
        return carry

    lax.fori_loop(0, 2048 // _L, _fill_z, 0)

    def _fill_one(i, carry):
        ones_v[pl.ds(i * _L, _L)] = jnp.ones((_L,), jnp.float32)
        return carry

    lax.fori_loop(0, _CH // _L, _fill_one, 0)
    pltpu.sync_copy(dst_hbm.at[pl.ds(w * _ECH, _ECH)], idx_v)

    @pl.when(s == 0)
    def _():
        def _z(i, carry):
            pltpu.sync_copy(zch_v, deg_sh.at[pl.ds(i * 2048, 2048)])
            return carry
        lax.fori_loop(0, _NPAD // 2048, _z, 0)

    plsc.subcore_barrier()

    def _chunk(i, carry):
        pltpu.sync_copy(ones_v, deg_sh.at[idx_v.at[i]], add=True)
        return carry

    lax.fori_loop(0, _ECH, _chunk, 0)
    plsc.subcore_barrier()

    @pl.when(s == 0)
    def _():
        pltpu.sync_copy(deg_sh, out_hbm.at[pl.ds(c * _NPAD, _NPAD)])


_deg_call = functools.partial(
    pl.kernel,
    out_type=jax.ShapeDtypeStruct((_NC * _NPAD,), jnp.float32),
    mesh=_mesh,
    scratch_types=[
        pltpu.VMEM((_ECH, _CH), jnp.int32),
        pltpu.VMEM((_CH,), jnp.float32),
        pltpu.VMEM((2048,), jnp.float32),
        pltpu.VMEM_SHARED((_NPAD,), jnp.float32),
    ],
)(_deg_body)


def _agg_body(hs_hbm, src_hbm, dst_hbm, out_hbm, isrc_v, id0, id1, rows0,
              rows1, acc_sh, gs0, gs1, ss0, ss1):
    c = lax.axis_index("c")
    s = lax.axis_index("s")
    w = _wid()

    def _zero_row(r, carry):
        for j in range(_D // _L):
            rows0[r, pl.ds(j * _L, _L)] = jnp.zeros((_L,), jnp.float32)
        return carry

    lax.fori_loop(0, _CH, _zero_row, 0)
    for k in range(_RPT // _CH):
        pltpu.sync_copy(rows0, acc_sh.at[pl.ds(s * _RPT + k * _CH, _CH)])
    base = jnp.where(c == 0, s * _N0, _N0 * _NS + s * _N1)
    nch = jnp.where(c == 0, _N0, _N1)
    pltpu.sync_copy(src_hbm.at[pl.ds(base, _NMX)], isrc_v)
    plsc.subcore_barrier()

    pltpu.async_copy(hs_hbm.at[isrc_v.at[0]], rows0, gs0)
    pltpu.async_copy(dst_hbm.at[base], id0, gs0)

    def _drain(rows, idx, sem):
        pltpu.make_async_copy(hs_hbm.at[isrc_v.at[0]], rows, sem).wait()
        pltpu.make_async_copy(dst_hbm.at[0], idx, sem).wait()

    def _body(ii, carry):
        i0 = ii * 2
        i1 = i0 + 1
        _drain(rows0, id0, gs0)

        @pl.when(ii > 0)
        def _():
            pltpu.make_async_copy(rows1, acc_sh.at[id1], ss1).wait()

        pltpu.async_copy(hs_hbm.at[isrc_v.at[i1]], rows1, gs1)
        pltpu.async_copy(dst_hbm.at[base + i1], id1, gs1)
        pltpu.async_copy(rows0, acc_sh.at[id0], ss0, add=True)
        _drain(rows1, id1, gs1)
        pltpu.make_async_copy(rows0, acc_sh.at[id0], ss0).wait()

        @pl.when(i1 + 1 < nch)
        def _():
            pltpu.async_copy(hs_hbm.at[isrc_v.at[i1 + 1]], rows0, gs0)
            pltpu.async_copy(dst_hbm.at[base + i1 + 1], id0, gs0)

        pltpu.async_copy(rows1, acc_sh.at[id1], ss1, add=True)
        return carry

    lax.fori_loop(0, nch // 2, _body, 0)
    pltpu.make_async_copy(rows1, acc_sh.at[id1], ss1).wait()
    plsc.subcore_barrier()

    for k in range(_RPT // _CH):
        r0 = s * _RPT + k * _CH
        pltpu.sync_copy(acc_sh.at[pl.ds(r0, _CH)],
                        out_hbm.at[pl.ds(c * _NPAD + r0, _CH)])


_agg_call = functools.partial(
    pl.kernel,
    out_type=jax.ShapeDtypeStruct((_NC * _NPAD, _D), jnp.float32),
    mesh=_mesh,
    scratch_types=[
        pltpu.VMEM((_NMX, _CH), jnp.int32),
        pltpu.VMEM((_CH,), jnp.int32),
        pltpu.VMEM((_CH,), jnp.int32),
        pltpu.VMEM((_CH, _D), jnp.float32),
        pltpu.VMEM((_CH, _D), jnp.float32),
        pltpu.VMEM_SHARED((_NPAD, _D), jnp.float32),
        pltpu.SemaphoreType.DMA,
        pltpu.SemaphoreType.DMA,
        pltpu.SemaphoreType.DMA,
        pltpu.SemaphoreType.DMA,
    ],
)(_agg_body)


def _link_body(zw_hbm, z_hbm, ai_hbm, bi_hbm, out_hbm, ia_v, ib_v, za0, zb0,
               za1, zb1, dots_v, gs0, gs1):
    c = lax.axis_index("c")
    s = lax.axis_index("s")
    base = jnp.where(c == 0, s * _L0, _L0 * _NS + s * _L1)
    nl = jnp.where(c == 0, _L0, _L1)
    pltpu.sync_copy(ai_hbm.at[pl.ds(base, _LMX)], ia_v)
    pltpu.sync_copy(bi_hbm.at[pl.ds(base, _LMX)], ib_v)
    pltpu.async_copy(zw_hbm.at[ia_v.at[0]], za0, gs0)
    pltpu.async_copy(z_hbm.at[ib_v.at[0]], zb0, gs0)

    def _compute(za_v, zb_v, i):
        def _pair(p, carry2):
            ps = [p * 4 + t for t in range(4)]
            accs = [za_v[q, pl.ds(0, _L)] * zb_v[q, pl.ds(0, _L)] for q in ps]
            for j in range(1, 2 * _H // _L):
                sl = pl.ds(j * _L, _L)
                accs = [a + za_v[q, sl] * zb_v[q, sl]
                        for a, q in zip(accs, ps)]
            for a, q in zip(accs, ps):
                dots_v[q, :] = a
            return carry2

        lax.fori_loop(0, _CHL // 4, _pair, 0)
        pltpu.sync_copy(dots_v, out_hbm.at[pl.ds((base + i) * _CHL, _CHL)])

    def _body(ii, carry):
        i0 = ii * 2
        i1 = i0 + 1
        pltpu.make_async_copy(zw_hbm.at[ia_v.at[0]], za0, gs0).wait()
        pltpu.make_async_copy(zw_hbm.at[ia_v.at[0]], zb0, gs0).wait()
        pltpu.async_copy(zw_hbm.at[ia_v.at[i1]], za1, gs1)
        pltpu.async_copy(z_hbm.at[ib_v.at[i1]], zb1, gs1)
        _compute(za0, zb0, i0)
        pltpu.make_async_copy(zw_hbm.at[ia_v.at[0]], za1, gs1).wait()
        pltpu.make_async_copy(zw_hbm.at[ia_v.at[0]], zb1, gs1).wait()

        @pl.when(i1 + 1 < nl)
        def _():
            pltpu.async_copy(zw_hbm.at[ia_v.at[i1 + 1]], za0, gs0)
            pltpu.async_copy(z_hbm.at[ib_v.at[i1 + 1]], zb0, gs0)

        _compute(za1, zb1, i1)
        return carry

    lax.fori_loop(0, nl // 2, _body, 0)


_link_call = functools.partial(
    pl.kernel,
    out_type=jax.ShapeDtypeStruct((_LPAD, _L), jnp.float32),
    mesh=_mesh,
    scratch_types=[
        pltpu.VMEM((_LMX, _CHL), jnp.int32),
        pltpu.VMEM((_LMX, _CHL), jnp.int32),
        pltpu.VMEM((_CHL, 2 * _H), jnp.float32),
        pltpu.VMEM((_CHL, 2 * _H), jnp.float32),
        pltpu.VMEM((_CHL, 2 * _H), jnp.float32),
        pltpu.VMEM((_CHL, 2 * _H), jnp.float32),
        pltpu.VMEM((_CHL, _L), jnp.float32),
        pltpu.SemaphoreType.DMA,
        pltpu.SemaphoreType.DMA,
    ],
)(_link_body)


def _dinv_of(d0_ref, d1_ref):
    d = d0_ref[0] + d1_ref[0] + 1.0
    return lax.rsqrt(d)


def _s1_body(x_ref, d0_ref, d1_ref, w1_ref, hs_ref):
    dinv = _dinv_of(d0_ref, d1_ref)
    h = jnp.dot(x_ref[...], w1_ref[...], preferred_element_type=jnp.float32)
    hs_ref[...] = h * dinv


def _s2_body(a0_ref, a1_ref, hs1_ref, d0_ref, d1_ref, b1_ref, w2_ref,
             h1_ref, hs2_ref):
    dinv = _dinv_of(d0_ref, d1_ref)
    t = (a0_ref[...] + a1_ref[...] + hs1_ref[...]) * dinv + b1_ref[...]
    h1 = jnp.maximum(t, 0.0)
    h1_ref[...] = h1
    hs2_ref[...] = jnp.dot(h1, w2_ref[...],
                           preferred_element_type=jnp.float32) * dinv


def _s3_body(a0_ref, a1_ref, hs2_ref, d0_ref, d1_ref, b2_ref, h1_ref, wp_ref,
             z_ref, zw_ref):
    dinv = _dinv_of(d0_ref, d1_ref)
    t = (a0_ref[...] + a1_ref[...] + hs2_ref[...]) * dinv + b2_ref[...]
    h2 = jnp.maximum(t, 0.0)
    zb = jnp.concatenate([h1_ref[...], h2], axis=1)
    z_ref[...] = zb
    zw_ref[...] = zb * wp_ref[...]


def _loss_body(dots_ref, bp_ref, out_ref):
    b = pl.program_id(0)
    blk = dots_ref[...]
    lane = lax.broadcasted_iota(jnp.int32, (128, 8), 0)
    grp = lax.broadcasted_iota(jnp.int32, (128, 8), 1)
    sel = jnp.where(lane // _L == grp, 1.0, 0.0)
    logit = jnp.dot(blk, sel, preferred_element_type=jnp.float32) + bp_ref[0, 0]
    row = lax.broadcasted_iota(jnp.int32, (_R, 8), 0)
    col = lax.broadcasted_iota(jnp.int32, (_R, 8), 1)
    pi = (b * _R + row) * 8 + col
    is_pos = pi < _P
    valid = is_pos | ((pi >= _PP) & (pi < _PP + _P))
    tgt = jnp.where(is_pos, 1.0, 0.0)
    term = (jnp.maximum(logit, 0.0) - logit * tgt
            + jnp.log1p(jnp.exp(-jnp.abs(logit))))
    term = jnp.where(valid, term, 0.0)

    @pl.when(b == 0)
    def _():
        out_ref[...] = jnp.zeros((1, 1), jnp.float32)

    out_ref[...] += jnp.sum(term).reshape(1, 1) / (2.0 * _P)


def _row_spec():
    return pl.BlockSpec((_R, _D), lambda i: (i, 0))


def _deg_spec():
    return pl.BlockSpec((1, _R, 1), lambda i: (i, 0, 0))


def _full_spec(shape):
    return pl.BlockSpec(shape, lambda i: tuple(0 for _ in shape))


def kernel(x, ei, pos, neg, W1, b1, W2, b2, Wp, bp):
    f32 = jnp.float32
    i32 = jnp.int32

    pad_idx = _NPAD - 1
    xp = jnp.concatenate([x, jnp.zeros((_NPAD - _N, _D), f32)], axis=0)
    epad2 = (_TCH + abs(_N0 - _N1)) * _CH
    srcp = jnp.concatenate([ei[0], jnp.full((epad2 - _E,), pad_idx, i32)])
    dstp = jnp.concatenate([ei[1], jnp.full((epad2 - _E,), pad_idx, i32)])
    srcp = srcp.reshape(-1, _CH)
    dstp = dstp.reshape(-1, _CH)
    fillp = jnp.full((_PP - _P,), pad_idx, i32)
    lslack = jnp.full((abs(_L0 - _L1) * _CHL,), pad_idx, i32)
    ai = jnp.concatenate([pos[0], fillp, neg[0], fillp, lslack]).reshape(-1, _CHL)
    bi = jnp.concatenate([pos[1], fillp, neg[1], fillp, lslack]).reshape(-1, _CHL)
    b1r = b1.reshape(1, _H)
    b2r = b2.reshape(1, _H)
    wpr = Wp.reshape(1, 2 * _H)
    bpr = bp.reshape(1, 1)

    degp = _deg_call(dstp).reshape(_NC, _NB, _R, 1)
    d0, d1 = degp[0], degp[1]

    hs1 = pl.pallas_call(
        _s1_body,
        grid=(_NB,),
        in_specs=[_row_spec(), _deg_spec(), _deg_spec(), _full_spec((_D, _H))],
        out_specs=_row_spec(),
        out_shape=jax.ShapeDtypeStruct((_NPAD, _H), f32),
    )(xp, d0, d1, W1)
    acc1 = _agg_call(hs1, srcp, dstp).reshape(_NC, _NPAD, _H)

    h1, hs2 = pl.pallas_call(
        _s2_body,
        grid=(_NB,),
        in_specs=[_row_spec(), _row_spec(), _row_spec(), _deg_spec(),
                  _deg_spec(), _full_spec((1, _H)), _full_spec((_H, _H))],
        out_specs=[_row_spec(), _row_spec()],
        out_shape=[jax.ShapeDtypeStruct((_NPAD, _H), f32),
                   jax.ShapeDtypeStruct((_NPAD, _H), f32)],
    )(acc1[0], acc1[1], hs1, d0, d1, b1r, W2)
    acc2 = _agg_call(hs2, srcp, dstp).reshape(_NC, _NPAD, _H)

    z, zw = pl.pallas_call(
        _s3_body,
        grid=(_NB,),
        in_specs=[_row_spec(), _row_spec(), _row_spec(), _deg_spec(),
                  _deg_spec(), _full_spec((1, _H)), _row_spec(),
                  _full_spec((1, 2 * _H))],
        out_specs=[pl.BlockSpec((_R, 2 * _H), lambda i: (i, 0)),
                   pl.BlockSpec((_R, 2 * _H), lambda i: (i, 0))],
        out_shape=[jax.ShapeDtypeStruct((_NPAD, 2 * _H), f32),
                   jax.ShapeDtypeStruct((_NPAD, 2 * _H), f32)],
    )(acc2[0], acc2[1], hs2, d0, d1, b2r, h1, wpr)

    dots = _link_call(zw, z, ai, bi)
    dots2d = dots.reshape(_LPAD * _L // 128, 128)

    res = pl.pallas_call(
        _loss_body,
        grid=(dots2d.shape[0] // _R,),
        in_specs=[pl.BlockSpec((_R, 128), lambda i: (i, 0)),
                  _full_spec((1, 1))],
        out_specs=pl.BlockSpec((1, 1), lambda i: (0, 0)),
        out_shape=jax.ShapeDtypeStruct((1, 1), f32),
    )(dots2d, bpr)
    return res[0, 0]


# speedup vs baseline: 1.1105x; 1.0004x over previous
import functools

import jax
import jax.numpy as jnp
from jax import lax
from jax.experimental import pallas as pl
from jax.experimental.pallas import tpu as pltpu
from jax.experimental.pallas import tpu_sc as plsc

_NC = 2
_NS = 16
_NW = _NC * _NS
_L = 16

_N = 10000
_D = 128
_H = 128
_E = 320000
_P = 160000

_NPAD = 10240
_EPAD = 327680
_PP = 163840
_LPAD = 2 * _PP
_CH = 128
_EPW = _EPAD // _NW
_ECH = _EPW // _CH
_TCH = _EPAD // _CH
_N0 = 80
_N1 = 2 * _ECH - _N0
_NMX = max(_N0, _N1)
_CHL = 64
_LPW = _LPAD // _NW
_LCH = _LPW // _CHL
_TLCH = _LPAD // _CHL
_L0 = 160
_L1 = 2 * _LCH - _L0
_LMX = max(_L0, _L1)
_RPT = _NPAD // _NS

_R = 512
_NB = _NPAD // _R

_mesh = plsc.VectorSubcoreMesh(
    core_axis_name="c", subcore_axis_name="s", num_cores=_NC, num_subcores=_NS)


def _wid():
    return lax.axis_index("s") * _NC + lax.axis_index("c")


def _deg_body(dst_hbm, out_hbm, idx_v, ones_v, zch_v, deg_sh):
    c = lax.axis_index("c")
    s = lax.axis_index("s")
    w = _wid()

    def _fill_z(i, carry):
        zch_v[pl.ds(i * _L, _L)] = jnp.zeros((_L,), jnp.float32)
        return carry

    lax.fori_loop(0, 2048 // _L, _fill_z, 0)

    def _fill_one(i, carry):
        ones_v[pl.ds(i * _L, _L)] = jnp.ones((_L,), jnp.float32)
        return carry

    lax.fori_loop(0, _CH // _L, _fill_one, 0)
    pltpu.sync_copy(dst_hbm.at[pl.ds(w * _ECH, _ECH)], idx_v)

    @pl.when(s == 0)
    def _():
        def _z(i, carry):
            pltpu.sync_copy(zch_v, deg_sh.at[pl.ds(i * 2048, 2048)])
            return carry
        lax.fori_loop(0, _NPAD // 2048, _z, 0)

    plsc.subcore_barrier()

    def _chunk(i, carry):
        pltpu.sync_copy(ones_v, deg_sh.at[idx_v.at[i]], add=True)
        return carry

    lax.fori_loop(0, _ECH, _chunk, 0)
    plsc.subcore_barrier()

    @pl.when(s == 0)
    def _():
        pltpu.sync_copy(deg_sh, out_hbm.at[pl.ds(c * _NPAD, _NPAD)])


_deg_call = functools.partial(
    pl.kernel,
    out_type=jax.ShapeDtypeStruct((_NC * _NPAD,), jnp.float32),
    mesh=_mesh,
    scratch_types=[
        pltpu.VMEM((_ECH, _CH), jnp.int32),
        pltpu.VMEM((_CH,), jnp.float32),
        pltpu.VMEM((2048,), jnp.float32),
        pltpu.VMEM_SHARED((_NPAD,), jnp.float32),
    ],
)(_deg_body)


def _agg_body(hs_hbm, src_hbm, dst_hbm, out_hbm, isrc_v, id0, id1, rows0,
              rows1, acc_sh, gs0, gs1, ss0, ss1):
    c = lax.axis_index("c")
    s = lax.axis_index("s")

    def _zero_row(r, carry):
        for j in range(_D // _L):
            rows0[r, pl.ds(j * _L, _L)] = jnp.zeros((_L,), jnp.float32)
        return carry

    lax.fori_loop(0, _CH, _zero_row, 0)
    for k in range(_RPT // _CH):
        pltpu.sync_copy(rows0, acc_sh.at[pl.ds(s * _RPT + k * _CH, _CH)])
    base = jnp.where(c == 0, s * _N0, _N0 * _NS + s * _N1)
    nch = jnp.where(c == 0, _N0, _N1)
    pltpu.sync_copy(src_hbm.at[pl.ds(base, _NMX)], isrc_v)
    plsc.subcore_barrier()

    pltpu.async_copy(hs_hbm.at[isrc_v.at[0]], rows0, gs0)
    pltpu.async_copy(dst_hbm.at[base], id0, gs0)

    def _drain(rows, idx, sem):
        pltpu.make_async_copy(hs_hbm.at[isrc_v.at[0]], rows, sem).wait()
        pltpu.make_async_copy(dst_hbm.at[0], idx, sem).wait()

    def _body(ii, carry):
        i0 = ii * 2
        i1 = i0 + 1
        _drain(rows0, id0, gs0)

        @pl.when(ii > 0)
        def _():
            pltpu.make_async_copy(rows1, acc_sh.at[id1], ss1).wait()

        pltpu.async_copy(hs_hbm.at[isrc_v.at[i1]], rows1, gs1)
        pltpu.async_copy(dst_hbm.at[base + i1], id1, gs1)
        pltpu.async_copy(rows0, acc_sh.at[id0], ss0, add=True)
        _drain(rows1, id1, gs1)
        pltpu.make_async_copy(rows0, acc_sh.at[id0], ss0).wait()

        @pl.when(i1 + 1 < nch)
        def _():
            pltpu.async_copy(hs_hbm.at[isrc_v.at[i1 + 1]], rows0, gs0)
            pltpu.async_copy(dst_hbm.at[base + i1 + 1], id0, gs0)

        pltpu.async_copy(rows1, acc_sh.at[id1], ss1, add=True)
        return carry

    lax.fori_loop(0, nch // 2, _body, 0)
    pltpu.make_async_copy(rows1, acc_sh.at[id1], ss1).wait()
    plsc.subcore_barrier()

    for k in range(_RPT // _CH):
        r0 = s * _RPT + k * _CH
        pltpu.sync_copy(acc_sh.at[pl.ds(r0, _CH)],
                        out_hbm.at[pl.ds(c * _NPAD + r0, _CH)])


_agg_call = functools.partial(
    pl.kernel,
    out_type=jax.ShapeDtypeStruct((_NC * _NPAD, _D), jnp.float32),
    mesh=_mesh,
    scratch_types=[
        pltpu.VMEM((_NMX, _CH), jnp.int32),
        pltpu.VMEM((_CH,), jnp.int32),
        pltpu.VMEM((_CH,), jnp.int32),
        pltpu.VMEM((_CH, _D), jnp.float32),
        pltpu.VMEM((_CH, _D), jnp.float32),
        pltpu.VMEM_SHARED((_NPAD, _D), jnp.float32),
        pltpu.SemaphoreType.DMA,
        pltpu.SemaphoreType.DMA,
        pltpu.SemaphoreType.DMA,
        pltpu.SemaphoreType.DMA,
    ],
)(_agg_body)


def _link_body(zw_hbm, z_hbm, ai_hbm, bi_hbm, out_hbm, ia_v, ib_v, za0, zb0,
               za1, zb1, dots_v, gs0, gs1):
    c = lax.axis_index("c")
    s = lax.axis_index("s")
    base = jnp.where(c == 0, s * _L0, _L0 * _NS + s * _L1)
    nl = jnp.where(c == 0, _L0, _L1)
    pltpu.sync_copy(ai_hbm.at[pl.ds(base, _LMX)], ia_v)
    pltpu.sync_copy(bi_hbm.at[pl.ds(base, _LMX)], ib_v)
    pltpu.async_copy(zw_hbm.at[ia_v.at[0]], za0, gs0)
    pltpu.async_copy(z_hbm.at[ib_v.at[0]], zb0, gs0)

    def _compute(za_v, zb_v, i):
        def _pair(p, carry2):
            ps = [p * 4 + t for t in range(4)]
            accs = [za_v[q, pl.ds(0, _L)] * zb_v[q, pl.ds(0, _L)] for q in ps]
            for j in range(1, 2 * _H // _L):
                sl = pl.ds(j * _L, _L)
                accs = [a + za_v[q, sl] * zb_v[q, sl]
                        for a, q in zip(accs, ps)]
            for a, q in zip(accs, ps):
                dots_v[q, :] = a
            return carry2

        lax.fori_loop(0, _CHL // 4, _pair, 0)
        pltpu.sync_copy(dots_v, out_hbm.at[pl.ds((base + i) * _CHL, _CHL)])

    def _body(ii, carry):
        i0 = ii * 2
        i1 = i0 + 1
        pltpu.make_async_copy(zw_hbm.at[ia_v.at[0]], za0, gs0).wait()
        pltpu.make_async_copy(zw_hbm.at[ia_v.at[0]], zb0, gs0).wait()
        pltpu.async_copy(zw_hbm.at[ia_v.at[i1]], za1, gs1)
        pltpu.async_copy(z_hbm.at[ib_v.at[i1]], zb1, gs1)
        _compute(za0, zb0, i0)
        pltpu.make_async_copy(zw_hbm.at[ia_v.at[0]], za1, gs1).wait()
        pltpu.make_async_copy(zw_hbm.at[ia_v.at[0]], zb1, gs1).wait()

        @pl.when(i1 + 1 < nl)
        def _():
            pltpu.async_copy(zw_hbm.at[ia_v.at[i1 + 1]], za0, gs0)
            pltpu.async_copy(z_hbm.at[ib_v.at[i1 + 1]], zb0, gs0)

        _compute(za1, zb1, i1)
        return carry

    lax.fori_loop(0, nl // 2, _body, 0)


_link_call = functools.partial(
    pl.kernel,
    out_type=jax.ShapeDtypeStruct((_LPAD, _L), jnp.float32),
    mesh=_mesh,
    scratch_types=[
        pltpu.VMEM((_LMX, _CHL), jnp.int32),
        pltpu.VMEM((_LMX, _CHL), jnp.int32),
        pltpu.VMEM((_CHL, 2 * _H), jnp.float32),
        pltpu.VMEM((_CHL, 2 * _H), jnp.float32),
        pltpu.VMEM((_CHL, 2 * _H), jnp.float32),
        pltpu.VMEM((_CHL, 2 * _H), jnp.float32),
        pltpu.VMEM((_CHL, _L), jnp.float32),
        pltpu.SemaphoreType.DMA,
        pltpu.SemaphoreType.DMA,
    ],
)(_link_body)


def _dinv_of(d0_ref, d1_ref):
    d = d0_ref[0] + d1_ref[0] + 1.0
    return lax.rsqrt(d)


def _s1_body(x_ref, d0_ref, d1_ref, w1_ref, hs_ref):
    dinv = _dinv_of(d0_ref, d1_ref)
    h = jnp.dot(x_ref[...], w1_ref[...], preferred_element_type=jnp.float32)
    hs_ref[...] = h * dinv


def _s2_body(a0_ref, a1_ref, hs1_ref, d0_ref, d1_ref, b1_ref, w2_ref,
             h1_ref, hs2_ref):
    dinv = _dinv_of(d0_ref, d1_ref)
    t = (a0_ref[...] + a1_ref[...] + hs1_ref[...]) * dinv + b1_ref[...]
    h1 = jnp.maximum(t, 0.0)
    h1_ref[...] = h1
    hs2_ref[...] = jnp.dot(h1, w2_ref[...],
                           preferred_element_type=jnp.float32) * dinv


def _s3_body(a0_ref, a1_ref, hs2_ref, d0_ref, d1_ref, b2_ref, h1_ref, wp_ref,
             z_ref, zw_ref):
    dinv = _dinv_of(d0_ref, d1_ref)
    t = (a0_ref[...] + a1_ref[...] + hs2_ref[...]) * dinv + b2_ref[...]
    h2 = jnp.maximum(t, 0.0)
    zb = jnp.concatenate([h1_ref[...], h2], axis=1)
    z_ref[...] = zb
    zw_ref[...] = zb * wp_ref[...]


def _loss_body(dots_ref, bp_ref, out_ref):
    b = pl.program_id(0)
    blk = dots_ref[...]
    lane = lax.broadcasted_iota(jnp.int32, (128, 8), 0)
    grp = lax.broadcasted_iota(jnp.int32, (128, 8), 1)
    sel = jnp.where(lane // _L == grp, 1.0, 0.0)
    logit = jnp.dot(blk, sel, preferred_element_type=jnp.float32) + bp_ref[0, 0]
    row = lax.broadcasted_iota(jnp.int32, (_R, 8), 0)
    col = lax.broadcasted_iota(jnp.int32, (_R, 8), 1)
    pi = (b * _R + row) * 8 + col
    is_pos = pi < _P
    valid = is_pos | ((pi >= _PP) & (pi < _PP + _P))
    tgt = jnp.where(is_pos, 1.0, 0.0)
    term = (jnp.maximum(logit, 0.0) - logit * tgt
            + jnp.log1p(jnp.exp(-jnp.abs(logit))))
    term = jnp.where(valid, term, 0.0)

    @pl.when(b == 0)
    def _():
        out_ref[...] = jnp.zeros((1, 1), jnp.float32)

    out_ref[...] += jnp.sum(term).reshape(1, 1) / (2.0 * _P)


def _row_spec():
    return pl.BlockSpec((_R, _D), lambda i: (i, 0))


def _deg_spec():
    return pl.BlockSpec((1, _R, 1), lambda i: (i, 0, 0))


def _full_spec(shape):
    return pl.BlockSpec(shape, lambda i: tuple(0 for _ in shape))


def kernel(x, ei, pos, neg, W1, b1, W2, b2, Wp, bp):
    f32 = jnp.float32
    i32 = jnp.int32

    pad_idx = _NPAD - 1
    xp = jnp.concatenate([x, jnp.zeros((_NPAD - _N, _D), f32)], axis=0)
    epad2 = (_TCH + abs(_N0 - _N1)) * _CH
    srcp = jnp.concatenate([ei[0], jnp.full((epad2 - _E,), pad_idx, i32)])
    dstp = jnp.concatenate([ei[1], jnp.full((epad2 - _E,), pad_idx, i32)])
    srcp = srcp.reshape(-1, _CH)
    dstp = dstp.reshape(-1, _CH)
    fillp = jnp.full((_PP - _P,), pad_idx, i32)
    lslack = jnp.full((abs(_L0 - _L1) * _CHL,), pad_idx, i32)
    ai = jnp.concatenate([pos[0], fillp, neg[0], fillp, lslack]).reshape(-1, _CHL)
    bi = jnp.concatenate([pos[1], fillp, neg[1], fillp, lslack]).reshape(-1, _CHL)
    b1r = b1.reshape(1, _H)
    b2r = b2.reshape(1, _H)
    wpr = Wp.reshape(1, 2 * _H)
    bpr = bp.reshape(1, 1)

    degp = _deg_call(dstp).reshape(_NC, _NB, _R, 1)
    d0, d1 = degp[0], degp[1]

    hs1 = pl.pallas_call(
        _s1_body,
        grid=(_NB,),
        in_specs=[_row_spec(), _deg_spec(), _deg_spec(), _full_spec((_D, _H))],
        out_specs=_row_spec(),
        out_shape=jax.ShapeDtypeStruct((_NPAD, _H), f32),
    )(xp, d0, d1, W1)
    acc1 = _agg_call(hs1, srcp, dstp).reshape(_NC, _NPAD, _H)

    h1, hs2 = pl.pallas_call(
        _s2_body,
        grid=(_NB,),
        in_specs=[_row_spec(), _row_spec(), _row_spec(), _deg_spec(),
                  _deg_spec(), _full_spec((1, _H)), _full_spec((_H, _H))],
        out_specs=[_row_spec(), _row_spec()],
        out_shape=[jax.ShapeDtypeStruct((_NPAD, _H), f32),
                   jax.ShapeDtypeStruct((_NPAD, _H), f32)],
    )(acc1[0], acc1[1], hs1, d0, d1, b1r, W2)
    acc2 = _agg_call(hs2, srcp, dstp).reshape(_NC, _NPAD, _H)

    z, zw = pl.pallas_call(
        _s3_body,
        grid=(_NB,),
        in_specs=[_row_spec(), _row_spec(), _row_spec(), _deg_spec(),
                  _deg_spec(), _full_spec((1, _H)), _row_spec(),
                  _full_spec((1, 2 * _H))],
        out_specs=[pl.BlockSpec((_R, 2 * _H), lambda i: (i, 0)),
                   pl.BlockSpec((_R, 2 * _H), lambda i: (i, 0))],
        out_shape=[jax.ShapeDtypeStruct((_NPAD, 2 * _H), f32),
                   jax.ShapeDtypeStruct((_NPAD, 2 * _H), f32)],
    )(acc2[0], acc2[1], hs2, d0, d1, b2r, h1, wpr)

    dots = _link_call(zw, z, ai, bi)
    dots2d = dots.reshape(_LPAD * _L // 128, 128)

    res = pl.pallas_call(
        _loss_body,
        grid=(dots2d.shape[0] // _R,),
        in_specs=[pl.BlockSpec((_R, 128), lambda i: (i, 0)),
                  _full_spec((1, 1))],
        out_specs=pl.BlockSpec((1, 1), lambda i: (0, 0)),
        out_shape=jax.ShapeDtypeStruct((1, 1), f32),
    )(dots2d, bpr)
    return res[0, 0]
